# baseline (speedup 1.0000x reference)
import functools

import jax
import jax.numpy as jnp
from jax import lax
from jax.experimental import pallas as pl
from jax.experimental.pallas import tpu as pltpu
from jax.experimental.pallas import tpu_sc as plsc

N = 10000
E = 320000
H = 128
G = 64
K = 8
D0 = 16

NC, NS = 2, 16
NW = NC * NS
EPW = E // NW
CA = 128
NCH = EPW // CA
TAIL = EPW - NCH * CA
NP = 10240
RPT = NP // NS

R = 2000
NBLK = N // R


@functools.cache
def _sc_deg():
  mesh = plsc.VectorSubcoreMesh(
      core_axis_name="c", subcore_axis_name="s", num_cores=NC, num_subcores=NS)
  scratch = [
      pltpu.VMEM((NP,), jnp.float32),
      pltpu.VMEM((EPW,), jnp.int32),
  ]

  def body(dstr, out, hist, didx):
    c = lax.axis_index("c")
    s = lax.axis_index("s")
    wid = c * NS + s

    pltpu.sync_copy(dstr.at[pl.ds(wid * EPW, EPW)], didx)

    zero = jnp.zeros((16,), jnp.float32)

    def zrow(i, _):
      hist[pl.ds(i * 16, 16)] = zero
      return 0

    lax.fori_loop(0, NP // 16, zrow, 0)

    ones16 = jnp.ones((16,), jnp.float32)

    def edges(i, _):
      d = didx[pl.ds(i * 16, 16)]
      plsc.addupdate_scatter(hist, [d], ones16)
      return 0

    lax.fori_loop(0, EPW // 16, edges, 0)

    pltpu.sync_copy(hist, out.at[pl.ds(wid * NP, NP)])

  return pl.kernel(
      body,
      out_type=jax.ShapeDtypeStruct((NW * NP,), jnp.float32),
      mesh=mesh,
      scratch_types=scratch,
      compiler_params=pltpu.CompilerParams(needs_layout_passes=False),
  )


@functools.cache
def _sc_agg(D):
  mesh = plsc.VectorSubcoreMesh(
      core_axis_name="c", subcore_axis_name="s", num_cores=NC, num_subcores=NS)
  scratch = [
      pltpu.VMEM_SHARED((NP, D), jnp.float32),
      pltpu.VMEM((CA,), jnp.int32),
      pltpu.VMEM((CA, D), jnp.float32),
      pltpu.SemaphoreType.DMA,
      pltpu.VMEM((EPW,), jnp.int32),
      pltpu.VMEM((CA,), jnp.int32),
      pltpu.VMEM((CA, D), jnp.float32),
      pltpu.SemaphoreType.DMA,
      pltpu.VMEM((TAIL,), jnp.int32),
      pltpu.VMEM((TAIL, D), jnp.float32),
  ]

  def body(*refs):
    (xt, srcr, dstr, out, acc, didx0, rows0, sem0, sidx, didx1, rows1, sem1,
     didxt, rowst) = refs
    c = lax.axis_index("c")
    s = lax.axis_index("s")
    wid = c * NS + s

    zero = jnp.zeros((16,), jnp.float32)

    def zrow(r, _):
      for k in range(D // 16):
        rows0[r, pl.ds(k * 16, 16)] = zero
      return 0

    lax.fori_loop(0, CA, zrow, 0)

    pltpu.sync_copy(srcr.at[pl.ds(wid * EPW, EPW)], sidx)

    def zslice(j, _):
      pltpu.sync_copy(rows0, acc.at[pl.ds(s * RPT + j * CA, CA)])
      return 0

    lax.fori_loop(0, RPT // CA, zslice, 0)

    plsc.subcore_barrier()

    def load(chunk, dbuf, rbuf, sem):
      base = wid * EPW + chunk * CA
      pltpu.async_copy(dstr.at[pl.ds(base, CA)], dbuf, sem)
      pltpu.async_copy(xt.at[sidx.at[pl.ds(chunk * CA, CA)]], rbuf, sem)

    def lwait(chunk, dbuf, rbuf, sem):
      base = wid * EPW + chunk * CA
      pltpu.make_async_copy(dstr.at[pl.ds(base, CA)], dbuf, sem).wait()
      pltpu.make_async_copy(
          xt.at[sidx.at[pl.ds(chunk * CA, CA)]], rbuf, sem).wait()

    load(0, didx0, rows0, sem0)

    def step(t, _):
      j0 = 2 * t
      load(j0 + 1, didx1, rows1, sem1)
      lwait(j0, didx0, rows0, sem0)
      pltpu.sync_copy(rows0, acc.at[didx0], add=True)

      @pl.when(j0 + 2 < NCH)
      def _():
        load(j0 + 2, didx0, rows0, sem0)

      lwait(j0 + 1, didx1, rows1, sem1)
      pltpu.sync_copy(rows1, acc.at[didx1], add=True)
      return 0

    lax.fori_loop(0, NCH // 2, step, 0)

    base = wid * EPW + NCH * CA
    pltpu.sync_copy(dstr.at[pl.ds(base, TAIL)], didxt)
    pltpu.async_copy(
        xt.at[sidx.at[pl.ds(NCH * CA, TAIL)]], rowst, sem0).wait()
    pltpu.sync_copy(rowst, acc.at[didxt], add=True)

    plsc.subcore_barrier()

    r0 = s * RPT
    pltpu.sync_copy(acc.at[pl.ds(r0, RPT)], out.at[c, pl.ds(r0, RPT)])

  return pl.kernel(
      body,
      out_type=jax.ShapeDtypeStruct((NC, NP, D), jnp.float32),
      mesh=mesh,
      scratch_types=scratch,
  )


def _prep(degp, x16, W1p):
  def body(dp_ref, x_ref, w1_ref, dis_ref, z_ref):
    deg = jnp.sum(dp_ref[...], axis=1, keepdims=True) + 1.0
    dis = lax.rsqrt(deg)
    dis_ref[...] = dis
    z_ref[...] = jnp.dot(x_ref[...] * dis, w1_ref[...],
                         preferred_element_type=jnp.float32)

  return pl.pallas_call(
      body,
      grid=(NBLK,),
      in_specs=[
          pl.BlockSpec((R, NW), lambda i: (i, 0)),
          pl.BlockSpec((R, D0), lambda i: (i, 0)),
          pl.BlockSpec((D0, H), lambda i: (0, 0)),
      ],
      out_specs=[
          pl.BlockSpec((R, 1), lambda i: (i, 0)),
          pl.BlockSpec((R, H), lambda i: (i, 0)),
      ],
      out_shape=[
          jax.ShapeDtypeStruct((N, 1), jnp.float32),
          jax.ShapeDtypeStruct((N, H), jnp.float32),
      ],
  )(degp, x16, W1p)


def _l1(p, xt0, dis, W1p, b1, W2):
  def body(p_ref, xt_ref, dis_ref, w1_ref, b1_ref, w2_ref, o_ref):
    pre = (p_ref[0] + p_ref[1] + xt_ref[...]) * dis_ref[...]
    h1 = jnp.dot(pre, w1_ref[...], preferred_element_type=jnp.float32)
    h1 = jnp.maximum(h1 + b1_ref[...], 0.0)
    y2 = jnp.dot(h1, w2_ref[...], preferred_element_type=jnp.float32)
    o_ref[...] = y2 * dis_ref[...]

  return pl.pallas_call(
      body,
      grid=(NBLK,),
      in_specs=[
          pl.BlockSpec((2, R, D0), lambda i: (0, i, 0)),
          pl.BlockSpec((R, D0), lambda i: (i, 0)),
          pl.BlockSpec((R, 1), lambda i: (i, 0)),
          pl.BlockSpec((D0, H), lambda i: (0, 0)),
          pl.BlockSpec((1, H), lambda i: (0, 0)),
          pl.BlockSpec((H, H), lambda i: (0, 0)),
      ],
      out_specs=pl.BlockSpec((R, H), lambda i: (i, 0)),
      out_shape=jax.ShapeDtypeStruct((N, H), jnp.float32),
  )(p, xt0, dis, W1p, b1, W2)


def _lmid(p, xt, dis, b, W):
  def body(p_ref, xt_ref, dis_ref, b_ref, w_ref, o_ref):
    h = (p_ref[0] + p_ref[1] + xt_ref[...]) * dis_ref[...]
    h = jnp.maximum(h + b_ref[...], 0.0)
    y = jnp.dot(h, w_ref[...], preferred_element_type=jnp.float32)
    o_ref[...] = y * dis_ref[...]

  return pl.pallas_call(
      body,
      grid=(NBLK,),
      in_specs=[
          pl.BlockSpec((2, R, H), lambda i: (0, i, 0)),
          pl.BlockSpec((R, H), lambda i: (i, 0)),
          pl.BlockSpec((R, 1), lambda i: (i, 0)),
          pl.BlockSpec((1, H), lambda i: (0, 0)),
          pl.BlockSpec((H, H), lambda i: (0, 0)),
      ],
      out_specs=pl.BlockSpec((R, H), lambda i: (i, 0)),
      out_shape=jax.ShapeDtypeStruct((N, H), jnp.float32),
  )(p, xt, dis, b, W)


def _l3pool(p, xt, dis, b, batch2d, Wl, bl):
  def body(p_ref, xt_ref, dis_ref, b_ref, bt_ref, wl_ref, bl_ref, o_ref,
           sums_ref, cnts_ref):
    i = pl.program_id(0)

    @pl.when(i == 0)
    def _():
      sums_ref[...] = jnp.zeros_like(sums_ref)
      cnts_ref[...] = jnp.zeros_like(cnts_ref)

    h = (p_ref[0] + p_ref[1] + xt_ref[...]) * dis_ref[...]
    h = jnp.maximum(h + b_ref[...], 0.0)
    gid = lax.broadcasted_iota(jnp.int32, (R, G), 1)
    onehot = (bt_ref[...] == gid).astype(jnp.float32)
    sums_ref[...] += lax.dot_general(
        onehot, h, (((0,), (0,)), ((), ())), preferred_element_type=jnp.float32)
    ones = jnp.ones((R, H), jnp.float32)
    cnts_ref[...] += lax.dot_general(
        onehot, ones, (((0,), (0,)), ((), ())),
        preferred_element_type=jnp.float32)

    @pl.when(i == NBLK - 1)
    def _():
      pooled = sums_ref[...] / jnp.maximum(cnts_ref[...], 1.0)
      logits = jnp.dot(pooled, wl_ref[...], preferred_element_type=jnp.float32)
      logits = logits + bl_ref[...]
      m = jnp.max(logits, axis=1, keepdims=True)
      e = jnp.exp(logits - m)
      o_ref[...] = e / jnp.sum(e, axis=1, keepdims=True)

  return pl.pallas_call(
      body,
      grid=(NBLK,),
      in_specs=[
          pl.BlockSpec((2, R, H), lambda i: (0, i, 0)),
          pl.BlockSpec((R, H), lambda i: (i, 0)),
          pl.BlockSpec((R, 1), lambda i: (i, 0)),
          pl.BlockSpec((1, H), lambda i: (0, 0)),
          pl.BlockSpec((R, 1), lambda i: (i, 0)),
          pl.BlockSpec((H, K), lambda i: (0, 0)),
          pl.BlockSpec((1, K), lambda i: (0, 0)),
      ],
      out_specs=pl.BlockSpec((G, K), lambda i: (0, 0)),
      out_shape=jax.ShapeDtypeStruct((G, K), jnp.float32),
      scratch_shapes=[
          pltpu.VMEM((G, H), jnp.float32),
          pltpu.VMEM((G, H), jnp.float32),
      ],
  )(p, xt, dis, b, batch2d, Wl, bl)


def kernel(atomic_numbers, pos, edge_index, batch, W1, b1, W2, b2, W3, b3,
           Wl, bl):
  ei = edge_index.astype(jnp.int32)
  src = ei[0]
  dst = ei[1]
  batch2d = batch.astype(jnp.int32).reshape(N, 1)
  x16 = jnp.concatenate(
      [atomic_numbers[:, None], pos, jnp.zeros((N, D0 - 4), jnp.float32)],
      axis=1)
  W1p = jnp.concatenate([W1, jnp.zeros((D0 - 4, H), W1.dtype)], axis=0)
  b1r, b2r, b3r = b1.reshape(1, H), b2.reshape(1, H), b3.reshape(1, H)
  blr = bl.reshape(1, K)

  degp = _sc_deg()(dst)
  degc = degp.reshape(NW, NP).T[:N]
  dis, z1 = _prep(degc, x16, W1p)
  agg1 = _sc_agg(H)(z1, src, dst)
  xt1 = _lmid(agg1, z1, dis, b1r, W2)
  agg2 = _sc_agg(H)(xt1, src, dst)
  xt2 = _lmid(agg2, xt1, dis, b2r, W3)
  agg3 = _sc_agg(H)(xt2, src, dst)
  out = _l3pool(agg3, xt2, dis, b3r, batch2d, Wl, blr)
  return out[:, :, None]


# speedup vs baseline: 1.0568x; 1.0568x over previous
import functools

import jax
import jax.numpy as jnp
from jax import lax
from jax.experimental import pallas as pl
from jax.experimental.pallas import tpu as pltpu
from jax.experimental.pallas import tpu_sc as plsc

N = 10000
E = 320000
H = 128
G = 64
K = 8
D0 = 16

NC, NS = 2, 16
NW = NC * NS
EPW = E // NW
CA = 80
NCH = EPW // CA
NP = 10240
RPT = NP // NS

R = 2000
NBLK = N // R


@functools.cache
def _sc_deg():
  mesh = plsc.VectorSubcoreMesh(
      core_axis_name="c", subcore_axis_name="s", num_cores=NC, num_subcores=NS)
  scratch = [
      pltpu.VMEM((NP,), jnp.float32),
      pltpu.VMEM((EPW,), jnp.int32),
  ]

  def body(dstr, out, hist, didx):
    c = lax.axis_index("c")
    s = lax.axis_index("s")
    wid = c * NS + s

    pltpu.sync_copy(dstr.at[pl.ds(wid * EPW, EPW)], didx)

    zero = jnp.zeros((16,), jnp.float32)

    def zrow(i, _):
      hist[pl.ds(i * 16, 16)] = zero
      return 0

    lax.fori_loop(0, NP // 16, zrow, 0)

    ones16 = jnp.ones((16,), jnp.float32)

    def edges(i, _):
      d = didx[pl.ds(i * 16, 16)]
      plsc.addupdate_scatter(hist, [d], ones16)
      return 0

    lax.fori_loop(0, EPW // 16, edges, 0)

    pltpu.sync_copy(hist, out.at[pl.ds(wid * NP, NP)])

  return pl.kernel(
      body,
      out_type=jax.ShapeDtypeStruct((NW * NP,), jnp.float32),
      mesh=mesh,
      scratch_types=scratch,
      compiler_params=pltpu.CompilerParams(needs_layout_passes=False),
  )


@functools.cache
def _sc_agg(D):
  mesh = plsc.VectorSubcoreMesh(
      core_axis_name="c", subcore_axis_name="s", num_cores=NC, num_subcores=NS)
  scratch = [
      pltpu.VMEM_SHARED((NP, D), jnp.float32),
      pltpu.VMEM((EPW,), jnp.int32),
      pltpu.VMEM((CA,), jnp.int32),
      pltpu.VMEM((CA,), jnp.int32),
      pltpu.VMEM((CA,), jnp.int32),
      pltpu.VMEM((CA, D), jnp.float32),
      pltpu.VMEM((CA, D), jnp.float32),
      pltpu.VMEM((CA, D), jnp.float32),
      pltpu.SemaphoreType.DMA,
      pltpu.SemaphoreType.DMA,
      pltpu.SemaphoreType.DMA,
      pltpu.SemaphoreType.DMA,
      pltpu.SemaphoreType.DMA,
      pltpu.SemaphoreType.DMA,
  ]

  def body(*refs):
    (xt, srcr, dstr, out, acc, sidx, di0, di1, di2, ro0, ro1, ro2,
     sl0, sl1, sl2, ss0, ss1, ss2) = refs
    di = (di0, di1, di2)
    ro = (ro0, ro1, ro2)
    sl = (sl0, sl1, sl2)
    ss = (ss0, ss1, ss2)
    c = lax.axis_index("c")
    s = lax.axis_index("s")
    wid = c * NS + s

    zero = jnp.zeros((16,), jnp.float32)

    def zrow(r, _):
      for k in range(D // 16):
        ro0[r, pl.ds(k * 16, 16)] = zero
      return 0

    lax.fori_loop(0, CA, zrow, 0)

    pltpu.sync_copy(srcr.at[pl.ds(wid * EPW, EPW)], sidx)

    def zslice(j, _):
      pltpu.sync_copy(ro0, acc.at[pl.ds(s * RPT + j * CA, CA)])
      return 0

    lax.fori_loop(0, RPT // CA, zslice, 0)

    plsc.subcore_barrier()

    def load(chunk, u):
      base = wid * EPW + chunk * CA
      pltpu.async_copy(dstr.at[pl.ds(base, CA)], di[u], sl[u])
      pltpu.async_copy(xt.at[sidx.at[pl.ds(chunk * CA, CA)]], ro[u], sl[u])

    def lwait(chunk, u):
      base = wid * EPW + chunk * CA
      pltpu.make_async_copy(dstr.at[pl.ds(base, CA)], di[u], sl[u]).wait()
      pltpu.make_async_copy(
          xt.at[sidx.at[pl.ds(chunk * CA, CA)]], ro[u], sl[u]).wait()

    def sdrain(u):
      pltpu.make_async_copy(xt.at[pl.ds(0, CA)], ro[u], ss[u]).wait()

    def substep(j, u):
      lwait(j, u)
      pltpu.async_copy(ro[u], acc.at[di[u]], ss[u], add=True)
      up = (u + 2) % 3

      @pl.when(j > 0)
      def _():
        sdrain(up)

      @pl.when(j + 2 < NCH)
      def _():
        load(j + 2, up)

    load(0, 0)
    load(1, 1)

    def step(t, _):
      for u in range(3):
        substep(3 * t + u, u)
      return 0

    lax.fori_loop(0, NCH // 3, step, 0)
    substep(NCH - 2, 0)
    substep(NCH - 1, 1)
    sdrain(1)

    plsc.subcore_barrier()

    r0 = s * RPT
    pltpu.sync_copy(acc.at[pl.ds(r0, RPT)], out.at[c, pl.ds(r0, RPT)])

  return pl.kernel(
      body,
      out_type=jax.ShapeDtypeStruct((NC, NP, D), jnp.float32),
      mesh=mesh,
      scratch_types=scratch,
  )


def _prep(degp, x16, W1p):
  def body(dp_ref, x_ref, w1_ref, dis_ref, z_ref):
    deg = jnp.sum(dp_ref[...], axis=1, keepdims=True) + 1.0
    dis = lax.rsqrt(deg)
    dis_ref[...] = dis
    z_ref[...] = jnp.dot(x_ref[...] * dis, w1_ref[...],
                         preferred_element_type=jnp.float32)

  return pl.pallas_call(
      body,
      grid=(NBLK,),
      in_specs=[
          pl.BlockSpec((R, NW), lambda i: (i, 0)),
          pl.BlockSpec((R, D0), lambda i: (i, 0)),
          pl.BlockSpec((D0, H), lambda i: (0, 0)),
      ],
      out_specs=[
          pl.BlockSpec((R, 1), lambda i: (i, 0)),
          pl.BlockSpec((R, H), lambda i: (i, 0)),
      ],
      out_shape=[
          jax.ShapeDtypeStruct((N, 1), jnp.float32),
          jax.ShapeDtypeStruct((N, H), jnp.float32),
      ],
  )(degp, x16, W1p)


def _lmid(p, xt, dis, b, W):
  def body(p_ref, xt_ref, dis_ref, b_ref, w_ref, o_ref):
    h = (p_ref[0] + p_ref[1] + xt_ref[...]) * dis_ref[...]
    h = jnp.maximum(h + b_ref[...], 0.0)
    y = jnp.dot(h, w_ref[...], preferred_element_type=jnp.float32)
    o_ref[...] = y * dis_ref[...]

  return pl.pallas_call(
      body,
      grid=(NBLK,),
      in_specs=[
          pl.BlockSpec((2, R, H), lambda i: (0, i, 0)),
          pl.BlockSpec((R, H), lambda i: (i, 0)),
          pl.BlockSpec((R, 1), lambda i: (i, 0)),
          pl.BlockSpec((1, H), lambda i: (0, 0)),
          pl.BlockSpec((H, H), lambda i: (0, 0)),
      ],
      out_specs=pl.BlockSpec((R, H), lambda i: (i, 0)),
      out_shape=jax.ShapeDtypeStruct((N, H), jnp.float32),
  )(p, xt, dis, b, W)


def _l3pool(p, xt, dis, b, batch2d, Wl, bl):
  def body(p_ref, xt_ref, dis_ref, b_ref, bt_ref, wl_ref, bl_ref, o_ref,
           sums_ref, cnts_ref):
    i = pl.program_id(0)

    @pl.when(i == 0)
    def _():
      sums_ref[...] = jnp.zeros_like(sums_ref)
      cnts_ref[...] = jnp.zeros_like(cnts_ref)

    h = (p_ref[0] + p_ref[1] + xt_ref[...]) * dis_ref[...]
    h = jnp.maximum(h + b_ref[...], 0.0)
    gid = lax.broadcasted_iota(jnp.int32, (R, G), 1)
    onehot = (bt_ref[...] == gid).astype(jnp.float32)
    sums_ref[...] += lax.dot_general(
        onehot, h, (((0,), (0,)), ((), ())), preferred_element_type=jnp.float32)
    ones = jnp.ones((R, H), jnp.float32)
    cnts_ref[...] += lax.dot_general(
        onehot, ones, (((0,), (0,)), ((), ())),
        preferred_element_type=jnp.float32)

    @pl.when(i == NBLK - 1)
    def _():
      pooled = sums_ref[...] / jnp.maximum(cnts_ref[...], 1.0)
      logits = jnp.dot(pooled, wl_ref[...], preferred_element_type=jnp.float32)
      logits = logits + bl_ref[...]
      m = jnp.max(logits, axis=1, keepdims=True)
      e = jnp.exp(logits - m)
      o_ref[...] = e / jnp.sum(e, axis=1, keepdims=True)

  return pl.pallas_call(
      body,
      grid=(NBLK,),
      in_specs=[
          pl.BlockSpec((2, R, H), lambda i: (0, i, 0)),
          pl.BlockSpec((R, H), lambda i: (i, 0)),
          pl.BlockSpec((R, 1), lambda i: (i, 0)),
          pl.BlockSpec((1, H), lambda i: (0, 0)),
          pl.BlockSpec((R, 1), lambda i: (i, 0)),
          pl.BlockSpec((H, K), lambda i: (0, 0)),
          pl.BlockSpec((1, K), lambda i: (0, 0)),
      ],
      out_specs=pl.BlockSpec((G, K), lambda i: (0, 0)),
      out_shape=jax.ShapeDtypeStruct((G, K), jnp.float32),
      scratch_shapes=[
          pltpu.VMEM((G, H), jnp.float32),
          pltpu.VMEM((G, H), jnp.float32),
      ],
  )(p, xt, dis, b, batch2d, Wl, bl)


def kernel(atomic_numbers, pos, edge_index, batch, W1, b1, W2, b2, W3, b3,
           Wl, bl):
  ei = edge_index.astype(jnp.int32)
  src = ei[0]
  dst = ei[1]
  batch2d = batch.astype(jnp.int32).reshape(N, 1)
  x16 = jnp.concatenate(
      [atomic_numbers[:, None], pos, jnp.zeros((N, D0 - 4), jnp.float32)],
      axis=1)
  W1p = jnp.concatenate([W1, jnp.zeros((D0 - 4, H), W1.dtype)], axis=0)
  b1r, b2r, b3r = b1.reshape(1, H), b2.reshape(1, H), b3.reshape(1, H)
  blr = bl.reshape(1, K)

  degp = _sc_deg()(dst)
  degc = degp.reshape(NW, NP).T[:N]
  dis, z1 = _prep(degc, x16, W1p)
  agg1 = _sc_agg(H)(z1, src, dst)
  xt1 = _lmid(agg1, z1, dis, b1r, W2)
  agg2 = _sc_agg(H)(xt1, src, dst)
  xt2 = _lmid(agg2, xt1, dis, b2r, W3)
  agg3 = _sc_agg(H)(xt2, src, dst)
  out = _l3pool(agg3, xt2, dis, b3r, batch2d, Wl, blr)
  return out[:, :, None]


# speedup vs baseline: 1.0587x; 1.0018x over previous
import functools

import jax
import jax.numpy as jnp
from jax import lax
from jax.experimental import pallas as pl
from jax.experimental.pallas import tpu as pltpu
from jax.experimental.pallas import tpu_sc as plsc

N = 10000
E = 320000
H = 128
G = 64
K = 8
D0 = 16

NC, NS = 2, 16
NW = NC * NS
EPW = E // NW
CA = 80
NCH = EPW // CA
NP = 10240
RPT = NP // NS

R = 2000
NBLK = N // R


@functools.cache
def _sc_deg():
  mesh = plsc.VectorSubcoreMesh(
      core_axis_name="c", subcore_axis_name="s", num_cores=NC, num_subcores=NS)
  scratch = [
      pltpu.VMEM((NP,), jnp.float32),
      pltpu.VMEM((EPW,), jnp.int32),
  ]

  def body(dstr, out, hist, didx):
    c = lax.axis_index("c")
    s = lax.axis_index("s")
    wid = c * NS + s

    pltpu.sync_copy(dstr.at[pl.ds(wid * EPW, EPW)], didx)

    zero = jnp.zeros((16,), jnp.float32)

    def zrow(i, _):
      hist[pl.ds(i * 16, 16)] = zero
      return 0

    lax.fori_loop(0, NP // 16, zrow, 0)

    ones16 = jnp.ones((16,), jnp.float32)

    def edges(i, _):
      d = didx[pl.ds(i * 16, 16)]
      plsc.addupdate_scatter(hist, [d], ones16)
      return 0

    lax.fori_loop(0, EPW // 16, edges, 0)

    pltpu.sync_copy(hist, out.at[pl.ds(wid * NP, NP)])

  return pl.kernel(
      body,
      out_type=jax.ShapeDtypeStruct((NW * NP,), jnp.float32),
      mesh=mesh,
      scratch_types=scratch,
      compiler_params=pltpu.CompilerParams(needs_layout_passes=False),
  )


@functools.cache
def _sc_agg(D):
  mesh = plsc.VectorSubcoreMesh(
      core_axis_name="c", subcore_axis_name="s", num_cores=NC, num_subcores=NS)
  scratch = [
      pltpu.VMEM_SHARED((NP, D), jnp.float32),
      pltpu.VMEM((EPW,), jnp.int32),
      pltpu.VMEM((CA,), jnp.int32),
      pltpu.VMEM((CA,), jnp.int32),
      pltpu.VMEM((CA,), jnp.int32),
      pltpu.VMEM((CA, D), jnp.float32),
      pltpu.VMEM((CA, D), jnp.float32),
      pltpu.VMEM((CA, D), jnp.float32),
      pltpu.SemaphoreType.DMA,
      pltpu.SemaphoreType.DMA,
      pltpu.SemaphoreType.DMA,
      pltpu.SemaphoreType.DMA,
      pltpu.SemaphoreType.DMA,
      pltpu.SemaphoreType.DMA,
  ]

  def body(*refs):
    (xt, srcr, dstr, out, acc, sidx, di0, di1, di2, ro0, ro1, ro2,
     sl0, sl1, sl2, ss0, ss1, ss2) = refs
    di = (di0, di1, di2)
    ro = (ro0, ro1, ro2)
    sl = (sl0, sl1, sl2)
    ss = (ss0, ss1, ss2)
    c = lax.axis_index("c")
    s = lax.axis_index("s")
    wid = c * NS + s

    zero = jnp.zeros((16,), jnp.float32)

    def zrow(r, _):
      for k in range(D // 16):
        ro2[r, pl.ds(k * 16, 16)] = zero
      return 0

    lax.fori_loop(0, CA, zrow, 0)

    pltpu.sync_copy(srcr.at[pl.ds(wid * EPW, EPW)], sidx)

    def load(chunk, u):
      base = wid * EPW + chunk * CA
      pltpu.async_copy(dstr.at[pl.ds(base, CA)], di[u], sl[u])
      pltpu.async_copy(xt.at[sidx.at[pl.ds(chunk * CA, CA)]], ro[u], sl[u])

    def lwait(chunk, u):
      base = wid * EPW + chunk * CA
      pltpu.make_async_copy(dstr.at[pl.ds(base, CA)], di[u], sl[u]).wait()
      pltpu.make_async_copy(
          xt.at[sidx.at[pl.ds(chunk * CA, CA)]], ro[u], sl[u]).wait()

    def sdrain(u):
      pltpu.make_async_copy(xt.at[pl.ds(0, CA)], ro[u], ss[u]).wait()

    def substep(j, u):
      lwait(j, u)
      pltpu.async_copy(ro[u], acc.at[di[u]], ss[u], add=True)
      up = (u + 2) % 3

      @pl.when(j > 0)
      def _():
        sdrain(up)

      @pl.when(j + 2 < NCH)
      def _():
        load(j + 2, up)

    load(0, 0)
    load(1, 1)

    def zslice(j, _):
      pltpu.sync_copy(ro2, acc.at[pl.ds(s * RPT + j * CA, CA)])
      return 0

    lax.fori_loop(0, RPT // CA, zslice, 0)

    plsc.subcore_barrier()

    def step(t, _):
      for u in range(3):
        substep(3 * t + u, u)
      return 0

    lax.fori_loop(0, NCH // 3, step, 0)
    substep(NCH - 2, 0)
    substep(NCH - 1, 1)
    sdrain(1)

    plsc.subcore_barrier()

    r0 = s * RPT
    pltpu.sync_copy(acc.at[pl.ds(r0, RPT)], out.at[c, pl.ds(r0, RPT)])

  return pl.kernel(
      body,
      out_type=jax.ShapeDtypeStruct((NC, NP, D), jnp.float32),
      mesh=mesh,
      scratch_types=scratch,
  )


def _prep(degp, a2d, pos, W1):
  def body(dp_ref, a_ref, p_ref, w1_ref, dis_ref, z_ref):
    deg = jnp.sum(dp_ref[...], axis=1, keepdims=True) + 1.0
    dis = lax.rsqrt(deg)
    dis_ref[...] = dis
    x4 = jnp.concatenate([a_ref[...], p_ref[...]], axis=1)
    z_ref[...] = jnp.dot(x4 * dis, w1_ref[...],
                         preferred_element_type=jnp.float32)

  return pl.pallas_call(
      body,
      grid=(NBLK,),
      in_specs=[
          pl.BlockSpec((R, NW), lambda i: (i, 0)),
          pl.BlockSpec((R, 1), lambda i: (i, 0)),
          pl.BlockSpec((R, 3), lambda i: (i, 0)),
          pl.BlockSpec((4, H), lambda i: (0, 0)),
      ],
      out_specs=[
          pl.BlockSpec((R, 1), lambda i: (i, 0)),
          pl.BlockSpec((R, H), lambda i: (i, 0)),
      ],
      out_shape=[
          jax.ShapeDtypeStruct((N, 1), jnp.float32),
          jax.ShapeDtypeStruct((N, H), jnp.float32),
      ],
  )(degp, a2d, pos, W1)


def _lmid(p, xt, dis, b, W):
  def body(p_ref, xt_ref, dis_ref, b_ref, w_ref, o_ref):
    h = (p_ref[0] + p_ref[1] + xt_ref[...]) * dis_ref[...]
    h = jnp.maximum(h + b_ref[...], 0.0)
    y = jnp.dot(h, w_ref[...], preferred_element_type=jnp.float32)
    o_ref[...] = y * dis_ref[...]

  return pl.pallas_call(
      body,
      grid=(NBLK,),
      in_specs=[
          pl.BlockSpec((2, R, H), lambda i: (0, i, 0)),
          pl.BlockSpec((R, H), lambda i: (i, 0)),
          pl.BlockSpec((R, 1), lambda i: (i, 0)),
          pl.BlockSpec((1, H), lambda i: (0, 0)),
          pl.BlockSpec((H, H), lambda i: (0, 0)),
      ],
      out_specs=pl.BlockSpec((R, H), lambda i: (i, 0)),
      out_shape=jax.ShapeDtypeStruct((N, H), jnp.float32),
  )(p, xt, dis, b, W)


def _l3pool(p, xt, dis, b, batch2d, Wl, bl):
  def body(p_ref, xt_ref, dis_ref, b_ref, bt_ref, wl_ref, bl_ref, o_ref,
           sums_ref, cnts_ref):
    i = pl.program_id(0)

    @pl.when(i == 0)
    def _():
      sums_ref[...] = jnp.zeros_like(sums_ref)
      cnts_ref[...] = jnp.zeros_like(cnts_ref)

    h = (p_ref[0] + p_ref[1] + xt_ref[...]) * dis_ref[...]
    h = jnp.maximum(h + b_ref[...], 0.0)
    gid = lax.broadcasted_iota(jnp.int32, (R, G), 1)
    onehot = (bt_ref[...] == gid).astype(jnp.float32)
    sums_ref[...] += lax.dot_general(
        onehot, h, (((0,), (0,)), ((), ())), preferred_element_type=jnp.float32)
    ones = jnp.ones((R, H), jnp.float32)
    cnts_ref[...] += lax.dot_general(
        onehot, ones, (((0,), (0,)), ((), ())),
        preferred_element_type=jnp.float32)

    @pl.when(i == NBLK - 1)
    def _():
      pooled = sums_ref[...] / jnp.maximum(cnts_ref[...], 1.0)
      logits = jnp.dot(pooled, wl_ref[...], preferred_element_type=jnp.float32)
      logits = logits + bl_ref[...]
      m = jnp.max(logits, axis=1, keepdims=True)
      e = jnp.exp(logits - m)
      o_ref[...] = e / jnp.sum(e, axis=1, keepdims=True)

  return pl.pallas_call(
      body,
      grid=(NBLK,),
      in_specs=[
          pl.BlockSpec((2, R, H), lambda i: (0, i, 0)),
          pl.BlockSpec((R, H), lambda i: (i, 0)),
          pl.BlockSpec((R, 1), lambda i: (i, 0)),
          pl.BlockSpec((1, H), lambda i: (0, 0)),
          pl.BlockSpec((R, 1), lambda i: (i, 0)),
          pl.BlockSpec((H, K), lambda i: (0, 0)),
          pl.BlockSpec((1, K), lambda i: (0, 0)),
      ],
      out_specs=pl.BlockSpec((G, K), lambda i: (0, 0)),
      out_shape=jax.ShapeDtypeStruct((G, K), jnp.float32),
      scratch_shapes=[
          pltpu.VMEM((G, H), jnp.float32),
          pltpu.VMEM((G, H), jnp.float32),
      ],
  )(p, xt, dis, b, batch2d, Wl, bl)


def kernel(atomic_numbers, pos, edge_index, batch, W1, b1, W2, b2, W3, b3,
           Wl, bl):
  ei = edge_index.astype(jnp.int32)
  src = ei[0]
  dst = ei[1]
  batch2d = batch.astype(jnp.int32).reshape(N, 1)
  a2d = atomic_numbers.reshape(N, 1)
  b1r, b2r, b3r = b1.reshape(1, H), b2.reshape(1, H), b3.reshape(1, H)
  blr = bl.reshape(1, K)

  degp = _sc_deg()(dst)
  degc = degp.reshape(NW, NP).T[:N]
  dis, z1 = _prep(degc, a2d, pos, W1)
  agg1 = _sc_agg(H)(z1, src, dst)
  xt1 = _lmid(agg1, z1, dis, b1r, W2)
  agg2 = _sc_agg(H)(xt1, src, dst)
  xt2 = _lmid(agg2, xt1, dis, b2r, W3)
  agg3 = _sc_agg(H)(xt2, src, dst)
  out = _l3pool(agg3, xt2, dis, b3r, batch2d, Wl, blr)
  return out[:, :, None]
